# baseline (speedup 1.0000x reference)
import functools

import numpy as np
import jax
import jax.numpy as jnp
from jax import lax
from jax.experimental import pallas as pl
from jax.experimental.pallas import tpu as pltpu
from jax.experimental.pallas import tpu_sc as plsc

_NW = 32
_NSC = 16
_L = 16
_NVEC = 25
_CHUNK = _L * _NVEC
_NBUF = 2


def _pos_encoding(max_pos, embed_dim):
    pos = np.arange(max_pos)[:, np.newaxis].astype(np.float32)
    i = np.arange(embed_dim)[np.newaxis, :].astype(np.float32)
    angle_rates = 1.0 / np.power(10000, 2 * (i // 2) / np.float32(embed_dim))
    angle_rads = pos * angle_rates
    angle_rads[:, 0::2] = np.sin(angle_rads[:, 0::2])
    angle_rads[:, 1::2] = np.cos(angle_rads[:, 1::2])
    return angle_rads


def kernel(x_in, table):
    B, S = x_in.shape
    V, D = table.shape
    total = B * S
    n_chunks = total // _CHUNK
    per_w = n_chunks // _NW
    rows_per_sc = V // _NSC

    idx3 = x_in.reshape(n_chunks, _NVEC, _L).astype(jnp.int32)
    tmpl_np = np.tile(_pos_encoding(S, D), (_CHUNK // S, 1))
    tmpl = jnp.asarray(tmpl_np, dtype=jnp.float32)

    mesh = plsc.VectorSubcoreMesh(core_axis_name="c", subcore_axis_name="s")

    @functools.partial(
        pl.kernel,
        mesh=mesh,
        compiler_params=pltpu.CompilerParams(use_tc_tiling_on_sc=False),
        out_type=jax.ShapeDtypeStruct((n_chunks, _CHUNK, D), jnp.float32),
        scratch_types=[
            [pltpu.VMEM((_NVEC, _L), jnp.int32) for _ in range(_NBUF)],
            [pltpu.VMEM((_CHUNK, D), jnp.float32) for _ in range(_NBUF)],
            pltpu.VMEM((_CHUNK, D), jnp.float32),
            pltpu.VMEM_SHARED((V, D), jnp.float32),
            [pltpu.SemaphoreType.DMA for _ in range(_NBUF)],
            [pltpu.SemaphoreType.DMA for _ in range(_NBUF)],
            [pltpu.SemaphoreType.DMA for _ in range(_NBUF)],
            pltpu.SemaphoreType.DMA,
        ],
    )
    def k(idx_hbm, table_hbm, tmpl_hbm, out_hbm,
          idx_v, buf_v, tmpl_v, table_s, sem_i, sem_g, sem_w, sem_t):
        sid = lax.axis_index("s")
        wid = sid * 2 + lax.axis_index("c")
        base = wid * per_w

        pltpu.async_copy(tmpl_hbm, tmpl_v, sem_t)
        row0 = sid * rows_per_sc
        pltpu.sync_copy(table_hbm.at[pl.ds(row0, rows_per_sc)],
                        table_s.at[pl.ds(row0, rows_per_sc)])
        pltpu.make_async_copy(tmpl_hbm, tmpl_v, sem_t).wait()
        plsc.subcore_barrier()

        def start_stage(c, p):
            pltpu.async_copy(idx_hbm.at[base + c], idx_v[p], sem_i[p])

        def fire_gathers(p):
            pltpu.make_async_copy(idx_hbm.at[base], idx_v[p], sem_i[p]).wait()
            for j in range(_NVEC):
                if p % 2 == 0:
                    pltpu.async_copy(
                        table_s.at[idx_v[p][j]],
                        buf_v[p].at[pl.ds(j * _L, _L)], sem_g[p])
                else:
                    pltpu.async_copy(
                        table_hbm.at[idx_v[p].at[j]],
                        buf_v[p].at[pl.ds(j * _L, _L)], sem_g[p])

        def drain_gathers(p):
            pltpu.make_async_copy(
                table_hbm.at[pl.ds(0, _CHUNK)], buf_v[p], sem_g[p]).wait()

        def finish_stage(c, p):
            drain_gathers(p)
            buf = buf_v[p]

            @plsc.parallel_loop(0, _CHUNK, 1, unroll=8)
            def _(r):
                buf[r] = buf[r] + tmpl_v[r]

            pltpu.async_copy(buf, out_hbm.at[base + c], sem_w[p])

        def wait_wb(p):
            pltpu.make_async_copy(buf_v[p], out_hbm.at[base], sem_w[p]).wait()

        n_rounds = per_w // _NBUF

        for p in range(_NBUF):
            start_stage(p, p)
        for p in range(_NBUF):
            fire_gathers(p)
            start_stage(p + _NBUF, p)

        def body(g, _):
            c0 = g * _NBUF
            for p in range(_NBUF):
                finish_stage(c0 + p, p)

                @pl.when(g < n_rounds - 1)
                def _(p=p):
                    wait_wb(p)
                    fire_gathers(p)

                @pl.when(g < n_rounds - 2)
                def _(p=p):
                    start_stage(c0 + p + 2 * _NBUF, p)

            return 0

        lax.fori_loop(0, n_rounds, body, 0)
        for p in range(_NBUF):
            wait_wb(p)

    out = k(idx3, table, tmpl)
    return out.reshape(B, S, D)


# speedup vs baseline: 1.0239x; 1.0239x over previous
import functools

import numpy as np
import jax
import jax.numpy as jnp
from jax import lax
from jax.experimental import pallas as pl
from jax.experimental.pallas import tpu as pltpu
from jax.experimental.pallas import tpu_sc as plsc

_NW = 32
_NSC = 16
_L = 16
_NVEC = 25
_CHUNK = _L * _NVEC
_NBUF = 2


def _pos_encoding(max_pos, embed_dim):
    pos = np.arange(max_pos)[:, np.newaxis].astype(np.float32)
    i = np.arange(embed_dim)[np.newaxis, :].astype(np.float32)
    angle_rates = 1.0 / np.power(10000, 2 * (i // 2) / np.float32(embed_dim))
    angle_rads = pos * angle_rates
    angle_rads[:, 0::2] = np.sin(angle_rads[:, 0::2])
    angle_rads[:, 1::2] = np.cos(angle_rads[:, 1::2])
    return angle_rads


def kernel(x_in, table):
    B, S = x_in.shape
    V, D = table.shape
    total = B * S
    n_chunks = total // _CHUNK
    per_w = n_chunks // _NW
    rows_per_sc = V // _NSC

    idx3 = x_in.reshape(n_chunks, _NVEC, _L).astype(jnp.int32)
    tmpl_np = np.tile(_pos_encoding(S, D), (_CHUNK // S, 1))
    tmpl = jnp.asarray(tmpl_np, dtype=jnp.float32)

    mesh = plsc.VectorSubcoreMesh(core_axis_name="c", subcore_axis_name="s")

    @functools.partial(
        pl.kernel,
        mesh=mesh,
        compiler_params=pltpu.CompilerParams(use_tc_tiling_on_sc=False),
        out_type=jax.ShapeDtypeStruct((n_chunks, _CHUNK, D), jnp.float32),
        scratch_types=[
            [pltpu.VMEM((_NVEC, _L), jnp.int32) for _ in range(_NBUF)],
            [pltpu.VMEM((_CHUNK, D), jnp.float32) for _ in range(_NBUF)],
            pltpu.VMEM((_CHUNK, D), jnp.float32),
            pltpu.VMEM_SHARED((V, D), jnp.float32),
            [pltpu.SemaphoreType.DMA for _ in range(_NBUF)],
            [pltpu.SemaphoreType.DMA for _ in range(_NBUF)],
            [pltpu.SemaphoreType.DMA for _ in range(_NBUF)],
            pltpu.SemaphoreType.DMA,
        ],
    )
    def k(idx_hbm, table_hbm, tmpl_hbm, out_hbm,
          idx_v, buf_v, tmpl_v, table_s, sem_i, sem_g, sem_w, sem_t):
        sid = lax.axis_index("s")
        wid = sid * 2 + lax.axis_index("c")
        base = wid * per_w

        pltpu.async_copy(tmpl_hbm, tmpl_v, sem_t)
        row0 = sid * rows_per_sc
        pltpu.sync_copy(table_hbm.at[pl.ds(row0, rows_per_sc)],
                        table_s.at[pl.ds(row0, rows_per_sc)])
        pltpu.make_async_copy(tmpl_hbm, tmpl_v, sem_t).wait()
        plsc.subcore_barrier()

        def start_stage(c, p):
            pltpu.async_copy(idx_hbm.at[base + c], idx_v[p], sem_i[p])

        def fire_gathers(p):
            pltpu.make_async_copy(idx_hbm.at[base], idx_v[p], sem_i[p]).wait()
            descs = []
            for j in range(_NVEC):
                iv = idx_v[p][j]
                descs.append(pltpu.async_copy(
                    table_s.at[iv], buf_v[p].at[pl.ds(j * _L, _L)],
                    sem_g[p],
                ))
            return descs

        def drain_gathers(p):
            pltpu.make_async_copy(
                table_hbm.at[pl.ds(0, _CHUNK)], buf_v[p], sem_g[p]).wait()

        def finish_stage(c, p):
            drain_gathers(p)
            buf = buf_v[p]

            @plsc.parallel_loop(0, _CHUNK, 1, unroll=16)
            def _(r):
                buf[r] = buf[r] + tmpl_v[r]

            pltpu.async_copy(buf, out_hbm.at[base + c], sem_w[p])

        def wait_wb(p):
            pltpu.make_async_copy(buf_v[p], out_hbm.at[base], sem_w[p]).wait()

        n_rounds = per_w // _NBUF

        for p in range(_NBUF):
            start_stage(p, p)
        for p in range(_NBUF):
            fire_gathers(p)
            start_stage(p + _NBUF, p)

        def body(g, _):
            c0 = g * _NBUF
            for p in range(_NBUF):
                finish_stage(c0 + p, p)

                @pl.when(g < n_rounds - 1)
                def _(p=p):
                    wait_wb(p)
                    fire_gathers(p)

                @pl.when(g < n_rounds - 2)
                def _(p=p):
                    start_stage(c0 + p + 2 * _NBUF, p)

            return 0

        lax.fori_loop(0, n_rounds, body, 0)
        for p in range(_NBUF):
            wait_wb(p)

    out = k(idx3, table, tmpl)
    return out.reshape(B, S, D)
